# Initial kernel scaffold; baseline (speedup 1.0000x reference)
import jax
import jax.numpy as jnp
from jax.experimental import pallas as pl


def kernel(z, W):
    raise NotImplementedError("write your pallas kernel here")


# speedup vs baseline: 1.7992x; 1.7992x over previous
import jax
import jax.numpy as jnp
from jax.experimental import pallas as pl

_T = 2048


def _vq_block(z_ref, w_ref, zsq_ref, wsq_ref, zq_ref, idx_ref):
    zb = z_ref[...]
    w = w_ref[...]
    s = jax.lax.dot_general(zb, w, (((1,), (1,)), ((), ())),
                            preferred_element_type=jnp.float32)
    d = (zsq_ref[...] + wsq_ref[...]) - 2.0 * s
    m = jnp.min(d, axis=1, keepdims=True)
    iota = jax.lax.broadcasted_iota(jnp.int32, d.shape, 1)
    idx = jnp.min(jnp.where(d == m, iota, d.shape[1]), axis=1, keepdims=True)
    idx_ref[...] = idx
    onehot = (iota == idx).astype(jnp.float32)
    zq_ref[...] = jnp.dot(onehot, w, preferred_element_type=jnp.float32)


def kernel(z, W):
    B, C, H, Wd = z.shape
    N = B * H * Wd
    K = W.shape[0]
    zf = z.reshape(B, C, H * Wd).transpose(0, 2, 1).reshape(N, C)
    zsq = jnp.sum(zf ** 2, axis=1, keepdims=True)
    wsq = jnp.sum(W ** 2, axis=1).reshape(1, K)
    zq_flat, idx = pl.pallas_call(
        _vq_block,
        grid=(N // _T,),
        in_specs=[pl.BlockSpec((_T, C), lambda i: (i, 0)),
                  pl.BlockSpec((K, C), lambda i: (0, 0)),
                  pl.BlockSpec((_T, 1), lambda i: (i, 0)),
                  pl.BlockSpec((1, K), lambda i: (0, 0))],
        out_specs=[pl.BlockSpec((_T, C), lambda i: (i, 0)),
                   pl.BlockSpec((_T, 1), lambda i: (i, 0))],
        out_shape=[jax.ShapeDtypeStruct((N, C), jnp.float32),
                   jax.ShapeDtypeStruct((N, 1), jnp.int32)],
    )(zf, W, zsq, wsq)
    zq = zq_flat.reshape(B, H * Wd, C).transpose(0, 2, 1).reshape(B, C, H, Wd)
    return zq, idx.reshape(N)


# speedup vs baseline: 1.8696x; 1.0392x over previous
import jax
import jax.numpy as jnp
from jax.experimental import pallas as pl

_BB = 2
_T = _BB * 1024


def _vq_block(z_ref, w_ref, zsq_ref, wsq_ref, zq_ref, idx_ref):
    zraw = z_ref[...]
    w = w_ref[...]
    zb = jnp.transpose(zraw, (0, 2, 1)).reshape(_T, zraw.shape[1])
    s = jax.lax.dot_general(zb, w, (((1,), (1,)), ((), ())),
                            preferred_element_type=jnp.float32)
    d = (zsq_ref[...] + wsq_ref[...]) - 2.0 * s
    m = jnp.min(d, axis=1, keepdims=True)
    iota = jax.lax.broadcasted_iota(jnp.int32, d.shape, 1)
    idx = jnp.min(jnp.where(d == m, iota, d.shape[1]), axis=1, keepdims=True)
    idx_ref[...] = idx
    onehot = (iota == idx).astype(jnp.float32)
    zq = jnp.dot(onehot, w, preferred_element_type=jnp.float32)
    zq_ref[...] = jnp.transpose(
        zq.reshape(_BB, 1024, zraw.shape[1]), (0, 2, 1))


def kernel(z, W):
    B, C, H, Wd = z.shape
    HW = H * Wd
    N = B * HW
    K = W.shape[0]
    z3 = z.reshape(B, C, HW)
    zsq = jnp.sum(jnp.transpose(z3, (0, 2, 1)).reshape(N, C) ** 2,
                  axis=1, keepdims=True)
    wsq = jnp.sum(W ** 2, axis=1).reshape(1, K)
    zq3, idx = pl.pallas_call(
        _vq_block,
        grid=(B // _BB,),
        in_specs=[pl.BlockSpec((_BB, C, HW), lambda i: (i, 0, 0)),
                  pl.BlockSpec((K, C), lambda i: (0, 0)),
                  pl.BlockSpec((_T, 1), lambda i: (i, 0)),
                  pl.BlockSpec((1, K), lambda i: (0, 0))],
        out_specs=[pl.BlockSpec((_BB, C, HW), lambda i: (i, 0, 0)),
                   pl.BlockSpec((_T, 1), lambda i: (i, 0))],
        out_shape=[jax.ShapeDtypeStruct((B, C, HW), jnp.float32),
                   jax.ShapeDtypeStruct((N, 1), jnp.int32)],
    )(z3, W, zsq, wsq)
    return zq3.reshape(B, C, H, Wd), idx.reshape(N)


# speedup vs baseline: 2.1463x; 1.1480x over previous
import jax
import jax.numpy as jnp
from jax.experimental import pallas as pl

_BB = 4
_T = _BB * 1024


def _vq_block(z_ref, w_ref, w2_ref, zsq_ref, wsq_ref, zq_ref, idx_ref):
    w = w_ref[...]
    w2 = w2_ref[...]
    K = w.shape[0]
    wsq = wsq_ref[...]
    for b in range(_BB):
        zt = z_ref[b]
        HW = zt.shape[1]
        zb = jnp.transpose(zt, (1, 0))
        s2 = jax.lax.dot_general(zb, w2, (((1,), (1,)), ((), ())),
                                 preferred_element_type=jnp.float32)
        zsq = zsq_ref[pl.ds(b * 1024, 1024), :]
        d = (zsq + wsq) - s2
        m = jnp.min(d, axis=1, keepdims=True)
        iota = jax.lax.broadcasted_iota(jnp.int32, d.shape, 1)
        idx = jnp.min(jnp.where(d == m, iota, K), axis=1, keepdims=True)
        idx_ref[pl.ds(b * 1024, 1024), :] = idx
        onehot = (iota == idx).astype(jnp.float32)
        zq = jnp.dot(onehot, w, preferred_element_type=jnp.float32)
        zq_ref[b] = jnp.transpose(zq, (1, 0))


def kernel(z, W):
    B, C, H, Wd = z.shape
    HW = H * Wd
    N = B * HW
    K = W.shape[0]
    z3 = z.reshape(B, C, HW)
    zsq = jnp.sum(jnp.transpose(z3, (0, 2, 1)).reshape(N, C) ** 2,
                  axis=1, keepdims=True)
    wsq = jnp.sum(W ** 2, axis=1).reshape(1, K)
    zq3, idx = pl.pallas_call(
        _vq_block,
        grid=(B // _BB,),
        in_specs=[pl.BlockSpec((_BB, C, HW), lambda i: (i, 0, 0)),
                  pl.BlockSpec((K, C), lambda i: (0, 0)),
                  pl.BlockSpec((K, C), lambda i: (0, 0)),
                  pl.BlockSpec((_T, 1), lambda i: (i, 0)),
                  pl.BlockSpec((1, K), lambda i: (0, 0))],
        out_specs=[pl.BlockSpec((_BB, C, HW), lambda i: (i, 0, 0)),
                   pl.BlockSpec((_T, 1), lambda i: (i, 0))],
        out_shape=[jax.ShapeDtypeStruct((B, C, HW), jnp.float32),
                   jax.ShapeDtypeStruct((N, 1), jnp.int32)],
    )(z3, W, W + W, zsq, wsq)
    return zq3.reshape(B, C, H, Wd), idx.reshape(N)


# speedup vs baseline: 2.6804x; 1.2488x over previous
import jax
import jax.numpy as jnp
from jax.experimental import pallas as pl

_BB = 4
_T = _BB * 1024


def _vq_block(z_ref, w_ref, w2_ref, zsq_ref, wsq_ref, zq_ref, idx_ref):
    w = w_ref[...]
    w2 = w2_ref[...]
    K = w.shape[0]
    wsq = wsq_ref[...]
    s2s = [jax.lax.dot_general(jnp.transpose(z_ref[b], (1, 0)), w2,
                               (((1,), (1,)), ((), ())),
                               preferred_element_type=jnp.float32)
           for b in range(_BB)]
    ds = [(zsq_ref[pl.ds(b * 1024, 1024), :] + wsq) - s2s[b]
          for b in range(_BB)]
    ms = [jnp.min(ds[b], axis=1, keepdims=True) for b in range(_BB)]
    iota = jax.lax.broadcasted_iota(jnp.int32, ds[0].shape, 1)
    idxs = [jnp.min(jnp.where(ds[b] == ms[b], iota, K), axis=1, keepdims=True)
            for b in range(_BB)]
    for b in range(_BB):
        idx_ref[pl.ds(b * 1024, 1024), :] = idxs[b]
    zqs = [jnp.dot((iota == idxs[b]).astype(jnp.float32), w,
                   preferred_element_type=jnp.float32) for b in range(_BB)]
    for b in range(_BB):
        zq_ref[b] = jnp.transpose(zqs[b], (1, 0))


def kernel(z, W):
    B, C, H, Wd = z.shape
    HW = H * Wd
    N = B * HW
    K = W.shape[0]
    z3 = z.reshape(B, C, HW)
    zsq = jnp.sum(jnp.transpose(z3, (0, 2, 1)).reshape(N, C) ** 2,
                  axis=1, keepdims=True)
    wsq = jnp.sum(W ** 2, axis=1).reshape(1, K)
    zq3, idx = pl.pallas_call(
        _vq_block,
        grid=(B // _BB,),
        in_specs=[pl.BlockSpec((_BB, C, HW), lambda i: (i, 0, 0)),
                  pl.BlockSpec((K, C), lambda i: (0, 0)),
                  pl.BlockSpec((K, C), lambda i: (0, 0)),
                  pl.BlockSpec((_T, 1), lambda i: (i, 0)),
                  pl.BlockSpec((1, K), lambda i: (0, 0))],
        out_specs=[pl.BlockSpec((_BB, C, HW), lambda i: (i, 0, 0)),
                   pl.BlockSpec((_T, 1), lambda i: (i, 0))],
        out_shape=[jax.ShapeDtypeStruct((B, C, HW), jnp.float32),
                   jax.ShapeDtypeStruct((N, 1), jnp.int32)],
    )(z3, W, W + W, zsq, wsq)
    return zq3.reshape(B, C, H, Wd), idx.reshape(N)
